# Initial kernel scaffold; baseline (speedup 1.0000x reference)
import jax
import jax.numpy as jnp
from jax.experimental import pallas as pl


def kernel(x, row, col, weights):
    raise NotImplementedError("write your pallas kernel here")


# speedup vs baseline: 2.2426x; 2.2426x over previous
import functools

import jax
import jax.numpy as jnp
from jax import lax
from jax.experimental import pallas as pl
from jax.experimental.pallas import tpu as pltpu
from jax.experimental.pallas import tpu_sc as plsc

N_A = 259200
N_B = 65160
NNZ = 4
BC = 128
DST_SHAPE = (181, 360)

CH = 128
N_PAD = 65536
NCHUNKS = N_PAD // CH


def _regrid_sc(xT, col_r, w_r):
    info = plsc.get_sparse_core_info()
    nc, ns = info.num_cores, info.num_subcores
    nw = nc * ns
    per_w = NCHUNKS // nw
    mesh = plsc.VectorSubcoreMesh(core_axis_name="c", subcore_axis_name="s")

    @functools.partial(
        pl.kernel,
        mesh=mesh,
        compiler_params=pltpu.CompilerParams(needs_layout_passes=False),
        out_type=jax.ShapeDtypeStruct((N_PAD, BC), jnp.float32),
        scratch_types=[
            pltpu.VMEM((NNZ, CH), jnp.int32),
            pltpu.VMEM((NNZ * CH,), jnp.float32),
            pltpu.VMEM((NNZ, CH, BC), jnp.float32),
            pltpu.VMEM((CH, BC), jnp.float32),
            pltpu.SemaphoreType.DMA,
        ],
    )
    def k(xT_h, col_h, w_h, out_h, idx_v, w_v, rows_v, acc_v, sem):
        wid = lax.axis_index("s") * nc + lax.axis_index("c")
        c0 = wid * per_w

        def chunk_body(ci, carry):
            c = c0 + ci
            pltpu.sync_copy(col_h.at[c], idx_v)
            pltpu.sync_copy(w_h.at[c], w_v)
            copies = [
                pltpu.async_copy(xT_h.at[idx_v.at[kk]], rows_v.at[kk], sem)
                for kk in range(NNZ)
            ]
            for cp in copies:
                cp.wait()

            def dst_body(j, carry2):
                jv = jnp.full((16,), j, jnp.int32)
                wvec = [
                    plsc.load_gather(w_v, [jv + (kk * CH)])
                    for kk in range(NNZ)
                ]
                for f in range(BC // 16):
                    sl = pl.ds(f * 16, 16)
                    r = wvec[0] * rows_v[0, j, sl]
                    for kk in range(1, NNZ):
                        r = r + wvec[kk] * rows_v[kk, j, sl]
                    acc_v[j, sl] = r
                return carry2

            lax.fori_loop(0, CH, dst_body, 0, unroll=2)
            pltpu.sync_copy(acc_v, out_h.at[pl.ds(c * CH, CH)])
            return carry

        lax.fori_loop(0, per_w, chunk_body, 0)

    return k(xT, col_r, w_r)


def kernel(x, row, col, weights):
    lead_shape = x.shape[:-2]
    x_flat = x.reshape(-1, N_A)
    xT = x_flat.T

    pad = N_PAD - N_B
    colp = jnp.concatenate([col, jnp.zeros((pad * NNZ,), jnp.int32)])
    wp = jnp.concatenate([weights, jnp.zeros((pad * NNZ,), jnp.float32)])
    col_r = colp.reshape(NCHUNKS, CH, NNZ).transpose(0, 2, 1)
    w_r = wp.reshape(NCHUNKS, CH, NNZ).transpose(0, 2, 1).reshape(NCHUNKS, NNZ * CH)

    yT = _regrid_sc(xT, col_r, w_r)
    y = yT[:N_B].T
    ny, nx = DST_SHAPE
    return y.reshape(*lead_shape, ny, nx)


# speedup vs baseline: 2.3230x; 1.0359x over previous
import functools

import jax
import jax.numpy as jnp
from jax import lax
from jax.experimental import pallas as pl
from jax.experimental.pallas import tpu as pltpu
from jax.experimental.pallas import tpu_sc as plsc

N_A = 259200
N_B = 65160
NNZ = 4
BC = 128
DST_SHAPE = (181, 360)

CH = 128
N_PAD = 65536
NCHUNKS = N_PAD // CH


def _regrid_sc(xT, col_r, w_r):
    info = plsc.get_sparse_core_info()
    nc, ns = info.num_cores, info.num_subcores
    nw = nc * ns
    per_w = NCHUNKS // nw
    wstride = NNZ * CH
    mesh = plsc.VectorSubcoreMesh(core_axis_name="c", subcore_axis_name="s")

    @functools.partial(
        pl.kernel,
        mesh=mesh,
        compiler_params=pltpu.CompilerParams(needs_layout_passes=False),
        out_type=jax.ShapeDtypeStruct((N_PAD, BC), jnp.float32),
        scratch_types=[
            pltpu.VMEM((per_w, NNZ, CH), jnp.int32),
            pltpu.VMEM((per_w * NNZ * CH,), jnp.float32),
            pltpu.VMEM((NNZ, CH, BC), jnp.float32),
            pltpu.VMEM((2, CH, BC), jnp.float32),
            pltpu.SemaphoreType.DMA,
            pltpu.SemaphoreType.DMA,
            pltpu.SemaphoreType.DMA,
            pltpu.SemaphoreType.DMA,
            pltpu.SemaphoreType.DMA,
            pltpu.SemaphoreType.DMA,
        ],
    )
    def k(xT_h, col_h, w_h, out_h, idx_v, w_v, rows_v, acc_v,
          g0, g1, g2, g3, s0, s1):
        gsem = (g0, g1, g2, g3)
        ssem = (s0, s1)
        wid = lax.axis_index("s") * nc + lax.axis_index("c")
        c0 = wid * per_w

        pltpu.sync_copy(col_h.at[pl.ds(c0, per_w)], idx_v)
        pltpu.sync_copy(w_h.at[pl.ds(c0 * wstride, per_w * wstride)], w_v)

        def fire(ci, kk):
            pltpu.async_copy(
                xT_h.at[idx_v.at[ci, kk]], rows_v.at[kk], gsem[kk])

        def gwait(ci, kk):
            pltpu.make_async_copy(
                xT_h.at[idx_v.at[ci, kk]], rows_v.at[kk], gsem[kk]).wait()

        for kk in range(NNZ):
            fire(0, kk)

        for ci in range(per_w):
            b = ci % 2
            if ci >= 2:
                pltpu.make_async_copy(
                    acc_v.at[b], out_h.at[pl.ds(0, CH)], ssem[b]).wait()
            for kk in range(NNZ):
                gwait(ci, kk)
                wbase = ci * wstride + kk * CH

                def dst_body(j, carry, _kk=kk, _b=b, _wbase=wbase):
                    wvec = plsc.load_gather(
                        w_v, [jnp.full((16,), j, jnp.int32) + _wbase])
                    for f in range(BC // 16):
                        sl = pl.ds(f * 16, 16)
                        r = wvec * rows_v[_kk, j, sl]
                        if _kk == 0:
                            acc_v[_b, j, sl] = r
                        else:
                            plsc.addupdate(acc_v.at[_b, j, sl], r)
                    return carry

                lax.fori_loop(0, CH, dst_body, 0, unroll=2)
                if ci + 1 < per_w:
                    fire(ci + 1, kk)
            pltpu.async_copy(
                acc_v.at[b], out_h.at[pl.ds((c0 + ci) * CH, CH)], ssem[b])

        for ci in (per_w - 2, per_w - 1):
            b = ci % 2
            pltpu.make_async_copy(
                acc_v.at[b], out_h.at[pl.ds(0, CH)], ssem[b]).wait()

    return k(xT, col_r, w_r)


def kernel(x, row, col, weights):
    lead_shape = x.shape[:-2]
    x_flat = x.reshape(-1, N_A)
    xT = x_flat.T

    pad = N_PAD - N_B
    colp = jnp.concatenate([col, jnp.zeros((pad * NNZ,), jnp.int32)])
    wp = jnp.concatenate([weights, jnp.zeros((pad * NNZ,), jnp.float32)])
    col_r = colp.reshape(NCHUNKS, CH, NNZ).transpose(0, 2, 1)
    w_r = wp.reshape(NCHUNKS, CH, NNZ).transpose(0, 2, 1).reshape(-1)

    yT = _regrid_sc(xT, col_r, w_r)
    y = yT[:N_B].T
    ny, nx = DST_SHAPE
    return y.reshape(*lead_shape, ny, nx)


# speedup vs baseline: 2.4657x; 1.0614x over previous
import functools

import jax
import jax.numpy as jnp
from jax import lax
from jax.experimental import pallas as pl
from jax.experimental.pallas import tpu as pltpu
from jax.experimental.pallas import tpu_sc as plsc

N_A = 259200
N_B = 65160
NNZ = 4
BC = 128
DST_SHAPE = (181, 360)

CH = 128
N_PAD = 65536
NCHUNKS = N_PAD // CH


def _regrid_sc(xT, col_r, w_r):
    info = plsc.get_sparse_core_info()
    nc, ns = info.num_cores, info.num_subcores
    nw = nc * ns
    per_w = NCHUNKS // nw
    wstride = NNZ * CH
    mesh = plsc.VectorSubcoreMesh(core_axis_name="c", subcore_axis_name="s")

    @functools.partial(
        pl.kernel,
        mesh=mesh,
        compiler_params=pltpu.CompilerParams(needs_layout_passes=False),
        out_type=jax.ShapeDtypeStruct((N_PAD, BC), jnp.float32),
        scratch_types=[
            pltpu.VMEM((per_w, NNZ, CH), jnp.int32),
            pltpu.VMEM((per_w * NNZ * CH,), jnp.float32),
            pltpu.VMEM((NNZ, CH, BC), jnp.float32),
            pltpu.VMEM((2, CH, BC), jnp.float32),
            pltpu.SemaphoreType.DMA,
            pltpu.SemaphoreType.DMA,
            pltpu.SemaphoreType.DMA,
            pltpu.SemaphoreType.DMA,
            pltpu.SemaphoreType.DMA,
            pltpu.SemaphoreType.DMA,
        ],
    )
    def k(xT_h, col_h, w_h, out_h, idx_v, w_v, rows_v, acc_v,
          g0, g1, g2, g3, s0, s1):
        gsem = (g0, g1, g2, g3)
        ssem = (s0, s1)
        wid = lax.axis_index("s") * nc + lax.axis_index("c")
        c0 = wid * per_w

        pltpu.sync_copy(col_h.at[pl.ds(c0, per_w)], idx_v)
        pltpu.sync_copy(w_h.at[pl.ds(c0 * wstride, per_w * wstride)], w_v)

        def fire(ci, kk):
            pltpu.async_copy(
                xT_h.at[idx_v.at[ci, kk]], rows_v.at[kk], gsem[kk])

        def gwait(ci, kk):
            pltpu.make_async_copy(
                xT_h.at[idx_v.at[ci, kk]], rows_v.at[kk], gsem[kk]).wait()

        for kk in range(NNZ):
            fire(0, kk)

        def pair_body(i, carry):
            for half in range(2):
                c = 2 * i + half

                @pl.when(c >= 2)
                def _wait_store(_b=half):
                    pltpu.make_async_copy(
                        acc_v.at[_b], out_h.at[pl.ds(0, CH)], ssem[_b]).wait()

                for kk in range(NNZ):
                    gwait(c, kk)
                    wbase = c * wstride + kk * CH

                    @plsc.parallel_loop(0, CH, 1, unroll=4)
                    def dst_body(j, _kk=kk, _b=half, _wbase=wbase):
                        wvec = plsc.load_gather(
                            w_v, [jnp.full((16,), j, jnp.int32) + _wbase])
                        for f in range(BC // 16):
                            sl = pl.ds(f * 16, 16)
                            r = wvec * rows_v[_kk, j, sl]
                            if _kk == 0:
                                acc_v[_b, j, sl] = r
                            else:
                                plsc.addupdate(acc_v.at[_b, j, sl], r)

                    @pl.when(c + 1 < per_w)
                    def _fire_next(_kk=kk, _c=c):
                        fire(_c + 1, _kk)

                pltpu.async_copy(
                    acc_v.at[half], out_h.at[pl.ds((c0 + c) * CH, CH)],
                    ssem[half])
            return carry

        lax.fori_loop(0, per_w // 2, pair_body, 0)

        for b in range(2):
            pltpu.make_async_copy(
                acc_v.at[b], out_h.at[pl.ds(0, CH)], ssem[b]).wait()

    return k(xT, col_r, w_r)


def kernel(x, row, col, weights):
    lead_shape = x.shape[:-2]
    x_flat = x.reshape(-1, N_A)
    xT = x_flat.T

    pad = N_PAD - N_B
    colp = jnp.concatenate([col, jnp.zeros((pad * NNZ,), jnp.int32)])
    wp = jnp.concatenate([weights, jnp.zeros((pad * NNZ,), jnp.float32)])
    col_r = colp.reshape(NCHUNKS, CH, NNZ).transpose(0, 2, 1)
    w_r = wp.reshape(NCHUNKS, CH, NNZ).transpose(0, 2, 1).reshape(-1)

    yT = _regrid_sc(xT, col_r, w_r)
    y = yT[:N_B].T
    ny, nx = DST_SHAPE
    return y.reshape(*lead_shape, ny, nx)


# speedup vs baseline: 3.4699x; 1.4073x over previous
import functools

import jax
import jax.numpy as jnp
from jax import lax
from jax.experimental import pallas as pl
from jax.experimental.pallas import tpu as pltpu
from jax.experimental.pallas import tpu_sc as plsc

N_A = 259200
N_B = 65160
NNZ = 4
BC = 128
DST_SHAPE = (181, 360)

CH = 64
N_PAD = 65536
NCHUNKS = N_PAD // CH


def _regrid_sc(xT, col_r, w_r):
    info = plsc.get_sparse_core_info()
    nc, ns = info.num_cores, info.num_subcores
    nw = nc * ns
    per_w = NCHUNKS // nw
    wstride = NNZ * CH
    mesh = plsc.VectorSubcoreMesh(core_axis_name="c", subcore_axis_name="s")

    @functools.partial(
        pl.kernel,
        mesh=mesh,
        compiler_params=pltpu.CompilerParams(needs_layout_passes=False),
        out_type=jax.ShapeDtypeStruct((N_PAD, BC), jnp.float32),
        scratch_types=[
            pltpu.VMEM((per_w, NNZ, CH), jnp.int32),
            pltpu.VMEM((per_w * NNZ * CH,), jnp.float32),
            pltpu.VMEM((2, NNZ, CH, BC), jnp.float32),
            pltpu.VMEM((2, CH, BC), jnp.float32),
            pltpu.SemaphoreType.DMA,
            pltpu.SemaphoreType.DMA,
            pltpu.SemaphoreType.DMA,
            pltpu.SemaphoreType.DMA,
        ],
    )
    def k(xT_h, col_h, w_h, out_h, idx_v, w_v, rows_v, acc_v,
          g0, g1, s0, s1):
        gsem = (g0, g1)
        ssem = (s0, s1)
        wid = lax.axis_index("s") * nc + lax.axis_index("c")
        c0 = wid * per_w

        pltpu.sync_copy(col_h.at[pl.ds(c0, per_w)], idx_v)
        pltpu.sync_copy(w_h.at[pl.ds(c0 * wstride, per_w * wstride)], w_v)

        def fire(ci, b):
            for kk in range(NNZ):
                pltpu.async_copy(
                    xT_h.at[idx_v.at[ci, kk]], rows_v.at[b, kk], gsem[b])

        def gwait(b):
            for kk in range(NNZ):
                pltpu.make_async_copy(
                    xT_h.at[idx_v.at[0, kk]], rows_v.at[b, kk], gsem[b]).wait()

        fire(0, 0)
        fire(1, 1)

        def pair_body(i, carry):
            for half in range(2):
                c = 2 * i + half

                @pl.when(c >= 2)
                def _wait_store(_b=half):
                    pltpu.make_async_copy(
                        acc_v.at[_b], out_h.at[pl.ds(0, CH)], ssem[_b]).wait()

                gwait(half)
                wbase = c * wstride

                @plsc.parallel_loop(0, CH, 1, unroll=4)
                def dst_body(j, _b=half, _wbase=wbase):
                    wsp = [
                        plsc.load_gather(
                            w_v,
                            [jnp.full((16,), kk * CH, jnp.int32) + (_wbase + j)])
                        for kk in range(NNZ)
                    ]
                    for f in range(BC // 16):
                        sl = pl.ds(f * 16, 16)
                        r01 = (wsp[0] * rows_v[_b, 0, j, sl]
                               + wsp[1] * rows_v[_b, 1, j, sl])
                        r23 = (wsp[2] * rows_v[_b, 2, j, sl]
                               + wsp[3] * rows_v[_b, 3, j, sl])
                        acc_v[_b, j, sl] = r01 + r23

                @pl.when(c + 2 < per_w)
                def _fire_next(_c=c, _b=half):
                    fire(_c + 2, _b)

                pltpu.async_copy(
                    acc_v.at[half], out_h.at[pl.ds((c0 + c) * CH, CH)],
                    ssem[half])
            return carry

        lax.fori_loop(0, per_w // 2, pair_body, 0)

        for b in range(2):
            pltpu.make_async_copy(
                acc_v.at[b], out_h.at[pl.ds(0, CH)], ssem[b]).wait()

    return k(xT, col_r, w_r)


def kernel(x, row, col, weights):
    lead_shape = x.shape[:-2]
    xT = jnp.transpose(x.reshape(-1, *x.shape[-2:]), (1, 2, 0)).reshape(N_A, -1)

    pad = N_PAD - N_B
    colp = jnp.concatenate([col, jnp.zeros((pad * NNZ,), jnp.int32)])
    wp = jnp.concatenate([weights, jnp.zeros((pad * NNZ,), jnp.float32)])
    col_r = colp.reshape(NCHUNKS, CH, NNZ).transpose(0, 2, 1)
    w_r = wp.reshape(NCHUNKS, CH, NNZ).transpose(0, 2, 1).reshape(-1)

    yT = _regrid_sc(xT, col_r, w_r)
    y = yT[:N_B].T
    ny, nx = DST_SHAPE
    return y.reshape(*lead_shape, ny, nx)
